# baseline (speedup 1.0000x reference)
import functools

import jax
import jax.numpy as jnp
from jax.experimental import pallas as pl
from jax.experimental.pallas import tpu as pltpu

_TB = 512
_GROUP = 8
_N = 16
_EMB = 128


def _nac_tile_kernel(adj0_ref, ops0_ref, adj1_ref, ops1_ref, mask_ref,
                     emb_ref, w0_ref, b0_ref, w1_ref, b1_ref,
                     fcw_ref, fcb_ref, out_ref):
    f32 = jnp.float32
    bf16 = jnp.bfloat16
    ngroups = _TB // _GROUP
    R = _TB * _N

    lane_r = jax.lax.broadcasted_iota(jnp.int32, (R, 128), 1)
    exp_lane = jax.lax.broadcasted_iota(jnp.int32, (_N, 128), 1)
    exp_sub = jax.lax.broadcasted_iota(jnp.int32, (_N, 128), 0)
    expander = ((exp_lane % _N) == exp_sub).astype(bf16)
    pool_sub = jax.lax.broadcasted_iota(jnp.int32, (_GROUP, 128), 0)
    pool_lane = jax.lax.broadcasted_iota(jnp.int32, (_GROUP, 128), 1)
    pool = ((pool_lane // _N) == pool_sub).astype(bf16) * bf16(1.0 / _N)

    mask = mask_ref[...]
    w1 = w1_ref[...]
    b0 = b0_ref[...]
    b1 = b1_ref[...]
    embw0 = jnp.dot(emb_ref[...], w0_ref[...],
                    preferred_element_type=f32).astype(bf16)

    def branch(adj_ref, ops_ref):
        oh3 = (ops_ref[...][:, :, None] ==
               jax.lax.broadcasted_iota(jnp.int32, (_TB, _N, 128), 2))
        oh = oh3.astype(bf16).reshape(R, 128)
        gw0 = jnp.dot(oh, embw0, preferred_element_type=f32).astype(bf16)
        a2 = adj_ref[...].astype(bf16).reshape(R, _N)
        bd = (jnp.dot(a2, expander,
                      preferred_element_type=f32).astype(bf16) * mask)
        x1s = []
        for g in range(ngroups):
            r0, r1 = g * 128, (g + 1) * 128
            x1s.append(jnp.maximum(
                jnp.dot(bd[r0:r1, :], gw0[r0:r1, :],
                        preferred_element_type=f32) + b0, 0.0).astype(bf16))
        ax1s = []
        for g in range(ngroups):
            r0, r1 = g * 128, (g + 1) * 128
            ax1s.append(jnp.dot(bd[r0:r1, :], x1s[g],
                                preferred_element_type=f32).astype(bf16))
        means = []
        for g in range(ngroups):
            x2 = jnp.maximum(
                jnp.dot(ax1s[g], w1, preferred_element_type=f32) + b1,
                0.0).astype(bf16)
            means.append(jnp.dot(pool, x2, preferred_element_type=f32))
        return jnp.concatenate(means, axis=0)

    m0 = branch(adj0_ref, ops0_ref)
    m1 = branch(adj1_ref, ops1_ref)
    fcw = fcw_ref[...]
    score = (jnp.dot(m0, fcw[:_EMB, :], preferred_element_type=f32)
             + jnp.dot(m1, fcw[_EMB:, :], preferred_element_type=f32)
             + fcb_ref[...])
    out_ref[...] = jax.nn.sigmoid(score)


@jax.jit
def kernel(matrix0, ops0, matrix1, ops1, emb, W0, b0, W1, b1, fc_w, fc_b):
    B, N, _ = matrix0.shape
    emb_pad = jnp.zeros((128, _EMB), jnp.float32).at[:emb.shape[0], :].set(emb)

    R = _TB * N
    rr = jnp.arange(R, dtype=jnp.int32)[:, None]
    cc = jnp.arange(128, dtype=jnp.int32)[None, :]
    mask = (((rr // N) % _GROUP) == (cc // N)).astype(jnp.bfloat16)

    grid = (B // _TB,)
    adj_spec = pl.BlockSpec((_TB, N, N), lambda i: (i, 0, 0))
    ops_spec = pl.BlockSpec((_TB, N), lambda i: (i, 0))
    full = lambda r, c: pl.BlockSpec((r, c), lambda i: (0, 0))

    out = pl.pallas_call(
        _nac_tile_kernel,
        grid=grid,
        in_specs=[adj_spec, ops_spec, adj_spec, ops_spec, full(R, 128),
                  full(128, _EMB), full(_EMB, _EMB), full(1, _EMB),
                  full(_EMB, _EMB), full(1, _EMB),
                  full(2 * _EMB, 1), full(1, 1)],
        out_specs=pl.BlockSpec((_TB, 1), lambda i: (i, 0)),
        out_shape=jax.ShapeDtypeStruct((B, 1), jnp.float32),
    )(matrix0, ops0, matrix1, ops1, mask, emb_pad, W0.astype(jnp.bfloat16),
      b0.reshape(1, _EMB), W1.astype(jnp.bfloat16), b1.reshape(1, _EMB),
      fc_w, fc_b.reshape(1, 1))
    return out.reshape(B)


# speedup vs baseline: 1.3418x; 1.3418x over previous
import functools

import jax
import jax.numpy as jnp
from jax.experimental import pallas as pl
from jax.experimental.pallas import tpu as pltpu

_TB = 512
_GROUP = 8
_N = 16
_EMB = 128


def _nac_tile_kernel(adj0_ref, ops0_ref, adj1_ref, ops1_ref, mask_ref,
                     emb_ref, w0_ref, b0_ref, w1_ref, b1_ref,
                     fcw_ref, fcb_ref, out_ref):
    f32 = jnp.float32
    bf16 = jnp.bfloat16
    ngroups = _TB // _GROUP
    R = _TB * _N

    lane_r = jax.lax.broadcasted_iota(jnp.int32, (R, 128), 1)
    exp_lane = jax.lax.broadcasted_iota(jnp.int32, (_N, 128), 1)
    exp_sub = jax.lax.broadcasted_iota(jnp.int32, (_N, 128), 0)
    expander = ((exp_lane % _N) == exp_sub).astype(bf16)
    pool_sub = jax.lax.broadcasted_iota(jnp.int32, (_GROUP, 128), 0)
    pool_lane = jax.lax.broadcasted_iota(jnp.int32, (_GROUP, 128), 1)
    pool = ((pool_lane // _N) == pool_sub).astype(bf16) * bf16(1.0 / _N)

    mask = mask_ref[...]
    w1 = w1_ref[...]
    b0 = b0_ref[...]
    b1 = b1_ref[...]
    embw0 = jnp.dot(emb_ref[...], w0_ref[...],
                    preferred_element_type=f32).astype(bf16)

    def branch(adj_ref, ops_ref):
        oh3 = (ops_ref[...][:, :, None] ==
               jax.lax.broadcasted_iota(jnp.int32, (_TB, _N, 128), 2))
        oh = oh3.astype(bf16).reshape(R, 128)
        gw0 = jnp.dot(oh, embw0, preferred_element_type=f32).astype(bf16)
        a2 = adj_ref[...].astype(bf16)
        bd = (jnp.dot(a2, expander,
                      preferred_element_type=f32).astype(bf16) * mask)
        x1s = []
        for g in range(ngroups):
            r0, r1 = g * 128, (g + 1) * 128
            x1s.append(jnp.maximum(
                jnp.dot(bd[r0:r1, :], gw0[r0:r1, :],
                        preferred_element_type=f32) + b0, 0.0).astype(bf16))
        ax1s = []
        for g in range(ngroups):
            r0, r1 = g * 128, (g + 1) * 128
            ax1s.append(jnp.dot(bd[r0:r1, :], x1s[g],
                                preferred_element_type=f32).astype(bf16))
        means = []
        for g in range(ngroups):
            x2 = jnp.maximum(
                jnp.dot(ax1s[g], w1, preferred_element_type=f32) + b1,
                0.0).astype(bf16)
            means.append(jnp.dot(pool, x2, preferred_element_type=f32))
        return jnp.concatenate(means, axis=0)

    m0 = branch(adj0_ref, ops0_ref)
    m1 = branch(adj1_ref, ops1_ref)
    fcw = fcw_ref[...]
    score = (jnp.dot(m0, fcw[:_EMB, :], preferred_element_type=f32)
             + jnp.dot(m1, fcw[_EMB:, :], preferred_element_type=f32)
             + fcb_ref[...])
    out_ref[...] = jax.nn.sigmoid(score)


@jax.jit
def kernel(matrix0, ops0, matrix1, ops1, emb, W0, b0, W1, b1, fc_w, fc_b):
    B, N, _ = matrix0.shape
    emb_pad = jnp.zeros((128, _EMB), jnp.float32).at[:emb.shape[0], :].set(emb)

    R = _TB * N
    rr = jnp.arange(R, dtype=jnp.int32)[:, None]
    cc = jnp.arange(128, dtype=jnp.int32)[None, :]
    mask = (((rr // N) % _GROUP) == (cc // N)).astype(jnp.bfloat16)

    grid = (B // _TB,)
    adj_spec = pl.BlockSpec((_TB * N, N), lambda i: (i, 0))
    ops_spec = pl.BlockSpec((_TB, N), lambda i: (i, 0))
    full = lambda r, c: pl.BlockSpec((r, c), lambda i: (0, 0))

    out = pl.pallas_call(
        _nac_tile_kernel,
        grid=grid,
        in_specs=[adj_spec, ops_spec, adj_spec, ops_spec, full(R, 128),
                  full(128, _EMB), full(_EMB, _EMB), full(1, _EMB),
                  full(_EMB, _EMB), full(1, _EMB),
                  full(2 * _EMB, 1), full(1, 1)],
        out_specs=pl.BlockSpec((_TB, 1), lambda i: (i, 0)),
        out_shape=jax.ShapeDtypeStruct((B, 1), jnp.float32),
    )(matrix0.reshape(B * N, N), ops0, matrix1.reshape(B * N, N), ops1,
      mask, emb_pad, W0.astype(jnp.bfloat16),
      b0.reshape(1, _EMB), W1.astype(jnp.bfloat16), b1.reshape(1, _EMB),
      fc_w, fc_b.reshape(1, 1))
    return out.reshape(B)


# speedup vs baseline: 1.3522x; 1.0078x over previous
import functools

import jax
import jax.numpy as jnp
from jax.experimental import pallas as pl
from jax.experimental.pallas import tpu as pltpu

_TB = 512
_GROUP = 8
_N = 16
_EMB = 128


def _nac_tile_kernel(adj0_ref, ops0_ref, adj1_ref, ops1_ref, mask_ref,
                     emb_ref, w0_ref, b0_ref, w1_ref, b1_ref,
                     fcw_ref, fcb_ref, out_ref):
    f32 = jnp.float32
    bf16 = jnp.bfloat16
    ngroups = _TB // _GROUP
    R = _TB * _N

    lane_r = jax.lax.broadcasted_iota(jnp.int32, (R, 128), 1)
    exp_lane = jax.lax.broadcasted_iota(jnp.int32, (_N, 128), 1)
    exp_sub = jax.lax.broadcasted_iota(jnp.int32, (_N, 128), 0)
    expander = ((exp_lane % _N) == exp_sub).astype(bf16)
    pool_sub = jax.lax.broadcasted_iota(jnp.int32, (_GROUP, 128), 0)
    pool_lane = jax.lax.broadcasted_iota(jnp.int32, (_GROUP, 128), 1)
    pool = ((pool_lane // _N) == pool_sub).astype(bf16) * bf16(1.0 / _N)

    mask = mask_ref[...]
    w1 = w1_ref[...]
    b0 = b0_ref[...]
    b1 = b1_ref[...]
    embw0 = jnp.dot(emb_ref[...], w0_ref[...],
                    preferred_element_type=f32).astype(bf16)

    def branch(adj_ref, ops_ref):
        oh3 = (ops_ref[...][:, :, None] ==
               jax.lax.broadcasted_iota(jnp.int32, (_TB, _N, 128), 2))
        oh = oh3.astype(bf16).reshape(R, 128)
        gw0 = jnp.dot(oh, embw0, preferred_element_type=f32).astype(bf16)
        a2 = adj_ref[...]
        bd = (jnp.dot(a2, expander,
                      preferred_element_type=f32).astype(bf16) * mask)
        x1s = []
        for g in range(ngroups):
            r0, r1 = g * 128, (g + 1) * 128
            x1s.append(jnp.maximum(
                jnp.dot(bd[r0:r1, :], gw0[r0:r1, :],
                        preferred_element_type=f32) + b0, 0.0).astype(bf16))
        ax1s = []
        for g in range(ngroups):
            r0, r1 = g * 128, (g + 1) * 128
            ax1s.append(jnp.dot(bd[r0:r1, :], x1s[g],
                                preferred_element_type=f32).astype(bf16))
        means = []
        for g in range(ngroups):
            x2 = jnp.maximum(
                jnp.dot(ax1s[g], w1, preferred_element_type=f32) + b1,
                0.0).astype(bf16)
            means.append(jnp.dot(pool, x2, preferred_element_type=f32))
        return jnp.concatenate(means, axis=0)

    m0 = branch(adj0_ref, ops0_ref)
    m1 = branch(adj1_ref, ops1_ref)
    fcw = fcw_ref[...]
    score = (jnp.dot(m0, fcw[:_EMB, :], preferred_element_type=f32)
             + jnp.dot(m1, fcw[_EMB:, :], preferred_element_type=f32)
             + fcb_ref[...])
    out_ref[...] = jax.nn.sigmoid(score)


@jax.jit
def kernel(matrix0, ops0, matrix1, ops1, emb, W0, b0, W1, b1, fc_w, fc_b):
    B, N, _ = matrix0.shape
    emb_pad = jnp.zeros((128, _EMB), jnp.float32).at[:emb.shape[0], :].set(emb)

    R = _TB * N
    rr = jnp.arange(R, dtype=jnp.int32)[:, None]
    cc = jnp.arange(128, dtype=jnp.int32)[None, :]
    mask = (((rr // N) % _GROUP) == (cc // N)).astype(jnp.bfloat16)

    grid = (B // _TB,)
    adj_spec = pl.BlockSpec((_TB * N, N), lambda i: (i, 0))
    ops_spec = pl.BlockSpec((_TB, N), lambda i: (i, 0))
    full = lambda r, c: pl.BlockSpec((r, c), lambda i: (0, 0))

    out = pl.pallas_call(
        _nac_tile_kernel,
        grid=grid,
        in_specs=[adj_spec, ops_spec, adj_spec, ops_spec, full(R, 128),
                  full(128, _EMB), full(_EMB, _EMB), full(1, _EMB),
                  full(_EMB, _EMB), full(1, _EMB),
                  full(2 * _EMB, 1), full(1, 1)],
        out_specs=pl.BlockSpec((_TB, 1), lambda i: (i, 0)),
        out_shape=jax.ShapeDtypeStruct((B, 1), jnp.float32),
    )(matrix0.reshape(B * N, N).astype(jnp.bfloat16), ops0,
      matrix1.reshape(B * N, N).astype(jnp.bfloat16), ops1,
      mask, emb_pad, W0.astype(jnp.bfloat16),
      b0.reshape(1, _EMB), W1.astype(jnp.bfloat16), b1.reshape(1, _EMB),
      fc_w, fc_b.reshape(1, 1))
    return out.reshape(B)


# speedup vs baseline: 1.3647x; 1.0093x over previous
import functools

import jax
import jax.numpy as jnp
from jax.experimental import pallas as pl
from jax.experimental.pallas import tpu as pltpu

_TB = 1024
_GROUP = 8
_N = 16
_EMB = 128


def _nac_tile_kernel(adj0_ref, ops0_ref, adj1_ref, ops1_ref, mask_ref,
                     emb_ref, w0_ref, b0_ref, w1_ref, b1_ref,
                     fcw_ref, fcb_ref, out_ref):
    f32 = jnp.float32
    bf16 = jnp.bfloat16
    ngroups = _TB // _GROUP
    R = _TB * _N

    lane_r = jax.lax.broadcasted_iota(jnp.int32, (R, 128), 1)
    exp_lane = jax.lax.broadcasted_iota(jnp.int32, (_N, 128), 1)
    exp_sub = jax.lax.broadcasted_iota(jnp.int32, (_N, 128), 0)
    expander = ((exp_lane % _N) == exp_sub).astype(bf16)
    pool_sub = jax.lax.broadcasted_iota(jnp.int32, (_GROUP, 128), 0)
    pool_lane = jax.lax.broadcasted_iota(jnp.int32, (_GROUP, 128), 1)
    pool = ((pool_lane // _N) == pool_sub).astype(bf16) * bf16(1.0 / _N)

    mask = mask_ref[...]
    w1 = w1_ref[...]
    b0 = b0_ref[...]
    b1 = b1_ref[...]
    embw0 = jnp.dot(emb_ref[...], w0_ref[...],
                    preferred_element_type=f32).astype(bf16)

    def branch(adj_ref, ops_ref):
        oh3 = (ops_ref[...][:, :, None] ==
               jax.lax.broadcasted_iota(jnp.int32, (_TB, _N, 128), 2))
        oh = oh3.astype(bf16).reshape(R, 128)
        gw0 = jnp.dot(oh, embw0, preferred_element_type=f32).astype(bf16)
        a2 = adj_ref[...]
        bd = (jnp.dot(a2, expander,
                      preferred_element_type=f32).astype(bf16) * mask)
        x1s = []
        for g in range(ngroups):
            r0, r1 = g * 128, (g + 1) * 128
            x1s.append(jnp.maximum(
                jnp.dot(bd[r0:r1, :], gw0[r0:r1, :],
                        preferred_element_type=f32) + b0, 0.0).astype(bf16))
        ax1s = []
        for g in range(ngroups):
            r0, r1 = g * 128, (g + 1) * 128
            ax1s.append(jnp.dot(bd[r0:r1, :], x1s[g],
                                preferred_element_type=f32).astype(bf16))
        means = []
        for g in range(ngroups):
            x2 = jnp.maximum(
                jnp.dot(ax1s[g], w1, preferred_element_type=f32) + b1,
                0.0).astype(bf16)
            means.append(jnp.dot(pool, x2, preferred_element_type=f32))
        return jnp.concatenate(means, axis=0)

    m0 = branch(adj0_ref, ops0_ref)
    m1 = branch(adj1_ref, ops1_ref)
    fcw = fcw_ref[...]
    score = (jnp.dot(m0, fcw[:_EMB, :], preferred_element_type=f32)
             + jnp.dot(m1, fcw[_EMB:, :], preferred_element_type=f32)
             + fcb_ref[...])
    out_ref[...] = jax.nn.sigmoid(score)


@jax.jit
def kernel(matrix0, ops0, matrix1, ops1, emb, W0, b0, W1, b1, fc_w, fc_b):
    B, N, _ = matrix0.shape
    emb_pad = jnp.zeros((128, _EMB), jnp.float32).at[:emb.shape[0], :].set(emb)

    R = _TB * N
    rr = jnp.arange(R, dtype=jnp.int32)[:, None]
    cc = jnp.arange(128, dtype=jnp.int32)[None, :]
    mask = (((rr // N) % _GROUP) == (cc // N)).astype(jnp.bfloat16)

    grid = (B // _TB,)
    adj_spec = pl.BlockSpec((_TB * N, N), lambda i: (i, 0))
    ops_spec = pl.BlockSpec((_TB, N), lambda i: (i, 0))
    full = lambda r, c: pl.BlockSpec((r, c), lambda i: (0, 0))

    out = pl.pallas_call(
        _nac_tile_kernel,
        grid=grid,
        in_specs=[adj_spec, ops_spec, adj_spec, ops_spec, full(R, 128),
                  full(128, _EMB), full(_EMB, _EMB), full(1, _EMB),
                  full(_EMB, _EMB), full(1, _EMB),
                  full(2 * _EMB, 1), full(1, 1)],
        out_specs=pl.BlockSpec((_TB, 1), lambda i: (i, 0)),
        out_shape=jax.ShapeDtypeStruct((B, 1), jnp.float32),
    )(matrix0.reshape(B * N, N).astype(jnp.bfloat16), ops0,
      matrix1.reshape(B * N, N).astype(jnp.bfloat16), ops1,
      mask, emb_pad, W0.astype(jnp.bfloat16),
      b0.reshape(1, _EMB), W1.astype(jnp.bfloat16), b1.reshape(1, _EMB),
      fc_w, fc_b.reshape(1, 1))
    return out.reshape(B)
